# baseline (speedup 1.0000x reference)
import functools

import jax
import jax.numpy as jnp
from jax import lax
from jax.experimental import pallas as pl
from jax.experimental.pallas import tpu as pltpu
from jax.experimental.pallas import tpu_sc as plsc

L = 16
NC = 2
NS = 16
ROW = 128


def _precompute_body(nblk, h_ref, r_ref, t_ref, m_ref, ts_ref, rs_ref):
    i = pl.program_id(0)
    ts_ref[...] = jnp.sum(t_ref[...], axis=1, keepdims=True)

    @pl.when(i == 0)
    def _():
        hmat = h_ref[...]
        rmat = r_ref[...]
        m_ref[...] = lax.dot_general(
            hmat, rmat, (((1,), (1,)), ((), ())),
            preferred_element_type=jnp.float32)
        rs_ref[...] = jnp.sum(rmat, axis=1, keepdims=True)


def _precompute(h_table, r_pad, t_table):
    nh = h_table.shape[0]
    nrp = r_pad.shape[0]
    nt = t_table.shape[0]
    dim = h_table.shape[1]
    tb = 1168
    nblk = (nt + tb - 1) // tb
    return pl.pallas_call(
        functools.partial(_precompute_body, nblk),
        grid=(nblk,),
        in_specs=[
            pl.BlockSpec((nh, dim), lambda i: (0, 0)),
            pl.BlockSpec((nrp, dim), lambda i: (0, 0)),
            pl.BlockSpec((tb, dim), lambda i: (i, 0)),
        ],
        out_specs=(
            pl.BlockSpec((nh, nrp), lambda i: (0, 0)),
            pl.BlockSpec((tb, 1), lambda i: (i, 0)),
            pl.BlockSpec((nrp, 1), lambda i: (0, 0)),
        ),
        out_shape=(
            jax.ShapeDtypeStruct((nh, nrp), jnp.float32),
            jax.ShapeDtypeStruct((nt, 1), jnp.float32),
            jax.ShapeDtypeStruct((nrp, 1), jnp.float32),
        ),
    )(h_table, r_pad, t_table)


def _combine_body(acc_ref, v_ref):
    a = acc_ref[...]
    den = a[0:1, :] + a[2:3, :]
    num = a[1:2, :] + a[3:4, :]
    v_ref[...] = jnp.where(den > 0.0, num / den, 0.0)


def _combine(acc):
    b = acc.shape[1]
    return pl.pallas_call(
        _combine_body,
        out_shape=jax.ShapeDtypeStruct((1, b), jnp.float32),
    )(acc)


def _sc_kernel(n_tok, b, nrp,
               seg_hbm, r_hbm, t_hbm, h_hbm, m_hbm, tsum_hbm, rsum_hbm,
               acc_hbm,
               h_v, tsum_v, rsum_v, seg_l, r_l, t_l,
               seg_v, idx_v, s_v, ex_v, exd_v, zero_v,
               den_sp, num_sp,
               sem_in, sem_ga, sem_gb, sem_s):
    cid = lax.axis_index("c")
    sid = lax.axis_index("s")
    tid = cid * NS + sid
    tokens_per_w = n_tok // (NC * NS)
    rows_per_w = tokens_per_w // ROW
    bpw = b // NS
    tok0 = tid * tokens_per_w
    grp = 8
    ngrp = rows_per_w // grp

    in_copies = [
        (seg_hbm.at[pl.ds(tok0, tokens_per_w)], seg_l),
        (r_hbm.at[pl.ds(tok0, tokens_per_w)], r_l),
        (t_hbm.at[pl.ds(tok0, tokens_per_w)], t_l),
        (h_hbm, h_v),
        (tsum_hbm, tsum_v),
        (rsum_hbm, rsum_v),
    ]
    for src, dst in in_copies:
        pltpu.async_copy(src, dst, sem_in)

    @pl.loop(0, bpw // L)
    def _zero(k):
        zero_v[pl.ds(k * L, L)] = jnp.zeros((L,), jnp.float32)

    pltpu.sync_copy(zero_v, den_sp.at[pl.ds(sid * bpw, bpw)])
    pltpu.sync_copy(zero_v, num_sp.at[pl.ds(sid * bpw, bpw)])

    for src, dst in in_copies:
        pltpu.make_async_copy(src, dst, sem_in).wait()

    plsc.subcore_barrier()

    def _mk_idx(g):
        @pl.loop(g * grp, g * grp + grp)
        def _idx(j):
            for u in range(ROW // L):
                sl = pl.ds(u * L, L)
                fl = pl.ds(j * ROW + u * L, L)
                seg16 = seg_l[fl]
                r16 = r_l[fl]
                h16 = plsc.load_gather(h_v, [seg16])
                seg_v[j, sl] = seg16
                idx_v[j, sl] = h16 * nrp + r16

    gsems = (sem_ga, sem_gb)

    def _fire_gathers(g, sem):
        @pl.loop(g * grp, g * grp + grp)
        def _f(j):
            pltpu.async_copy(m_hbm.at[idx_v.at[j]], s_v.at[j], sem)

    def _drain_gathers(g, sem):
        @pl.loop(g * grp, g * grp + grp)
        def _d(j):
            pltpu.make_async_copy(m_hbm.at[idx_v.at[j]], s_v.at[j], sem).wait()

    _mk_idx(0)
    _fire_gathers(0, gsems[0])
    if ngrp > 1:
        _mk_idx(1)
    for g in range(ngrp):
        if g + 1 < ngrp:
            _fire_gathers(g + 1, gsems[(g + 1) % 2])
        _drain_gathers(g, gsems[g % 2])

        @pl.loop(g * grp, g * grp + grp)
        def _compute(j):
            for u in range(ROW // L):
                sl = pl.ds(u * L, L)
                fl = pl.ds(j * ROW + u * L, L)
                s16 = s_v[j, sl]
                t16 = t_l[fl]
                r16 = r_l[fl]
                d16 = (plsc.load_gather(tsum_v, [t16])
                       - plsc.load_gather(rsum_v, [r16]))
                ex16 = jnp.exp(s16)
                ex_v[j, sl] = ex16
                exd_v[j, sl] = ex16 * d16

        @pl.loop(g * grp, g * grp + grp)
        def _scatter(j):
            pltpu.async_copy(ex_v.at[j], den_sp.at[seg_v.at[j]], sem_s,
                             add=True)
            pltpu.async_copy(exd_v.at[j], num_sp.at[seg_v.at[j]], sem_s,
                             add=True)

        if g + 2 < ngrp:
            _mk_idx(g + 2)

    @pl.loop(0, rows_per_w)
    def _drain_s(j):
        pltpu.make_async_copy(ex_v.at[j], den_sp.at[seg_v.at[j]],
                              sem_s).wait()
        pltpu.make_async_copy(exd_v.at[j], num_sp.at[seg_v.at[j]],
                              sem_s).wait()

    plsc.subcore_barrier()

    pltpu.sync_copy(den_sp.at[pl.ds(sid * bpw, bpw)],
                    acc_hbm.at[2 * cid, pl.ds(sid * bpw, bpw)])
    pltpu.sync_copy(num_sp.at[pl.ds(sid * bpw, bpw)],
                    acc_hbm.at[2 * cid + 1, pl.ds(sid * bpw, bpw)])


def _sc_run(seg, r_flat, t_flat, h, m_flat, tsum, rsum):
    n_tok = seg.shape[0]
    b = h.shape[0]
    ntp = tsum.shape[0]
    nrp = rsum.shape[0]
    bpw = b // NS
    tokens_per_w = n_tok // (NC * NS)
    rows_per_w = tokens_per_w // ROW
    mesh = plsc.VectorSubcoreMesh(
        core_axis_name="c", subcore_axis_name="s", num_cores=NC)
    grid_kernel = pl.kernel(
        functools.partial(_sc_kernel, n_tok, b, nrp),
        out_type=jax.ShapeDtypeStruct((4, b), jnp.float32),
        mesh=mesh,
        compiler_params=pltpu.CompilerParams(needs_layout_passes=False),
        scratch_types=[
            pltpu.VMEM((b,), jnp.int32),
            pltpu.VMEM((ntp,), jnp.float32),
            pltpu.VMEM((nrp,), jnp.float32),
            pltpu.VMEM((tokens_per_w,), jnp.int32),
            pltpu.VMEM((tokens_per_w,), jnp.int32),
            pltpu.VMEM((tokens_per_w,), jnp.int32),
            pltpu.VMEM((rows_per_w, ROW), jnp.int32),
            pltpu.VMEM((rows_per_w, ROW), jnp.int32),
            pltpu.VMEM((rows_per_w, ROW), jnp.float32),
            pltpu.VMEM((rows_per_w, ROW), jnp.float32),
            pltpu.VMEM((rows_per_w, ROW), jnp.float32),
            pltpu.VMEM((bpw,), jnp.float32),
            pltpu.VMEM_SHARED((b,), jnp.float32),
            pltpu.VMEM_SHARED((b,), jnp.float32),
            pltpu.SemaphoreType.DMA,
            pltpu.SemaphoreType.DMA,
            pltpu.SemaphoreType.DMA,
            pltpu.SemaphoreType.DMA,
        ],
    )
    return grid_kernel(seg, r_flat, t_flat, h, m_flat, tsum, rsum)


def kernel(h, r_flat, t_flat, segment_ids, H_table, R_table, T_table):
    b = h.shape[0]
    dim = H_table.shape[1]
    nr = R_table.shape[0]
    nrp = 64

    r_pad = jnp.pad(R_table, ((0, nrp - nr), (0, 0)))
    m, ts, rs = _precompute(H_table, r_pad, T_table)

    acc = _sc_run(segment_ids, r_flat, t_flat, h, m.reshape(-1),
                  ts.reshape(-1), rs.reshape(-1))
    v = _combine(acc)
    return jnp.broadcast_to(v.reshape(b, 1), (b, dim))


# speedup vs baseline: 1.6331x; 1.6331x over previous
import functools

import jax
import jax.numpy as jnp
from jax import lax
from jax.experimental import pallas as pl
from jax.experimental.pallas import tpu as pltpu
from jax.experimental.pallas import tpu_sc as plsc

L = 16
NC = 2
NS = 16
ROW = 128


def _precompute_body(nhp, ntp, nrp, h_ref, r_ref, t_ref, m_ref, ts_ref,
                     rs_ref):
    h_t = h_ref[...]
    r_raw = r_ref[...]
    r_t = jnp.concatenate(
        [r_raw, jnp.zeros((r_raw.shape[0], nrp - r_raw.shape[1]),
                          jnp.float32)], axis=1)
    t_t = t_ref[...]
    nh = h_t.shape[1]
    nt = t_t.shape[1]
    mm = lax.dot_general(h_t, r_t, (((0,), (0,)), ((), ())),
                         preferred_element_type=jnp.float32)
    m_ref[...] = jnp.concatenate(
        [mm, jnp.zeros((nhp - nh, mm.shape[1]), jnp.float32)], axis=0)
    ts = jnp.sum(t_t, axis=0, keepdims=True)
    ts_ref[...] = jnp.concatenate(
        [ts, jnp.zeros((1, ntp - nt), jnp.float32)], axis=1)
    rs_ref[...] = jnp.sum(r_t, axis=0, keepdims=True)


def _precompute(h_t, r_t, t_t):
    nh = h_t.shape[1]
    nrp = 128
    nt = t_t.shape[1]
    nhp = (nh + 7) // 8 * 8
    ntp = (nt + 1023) // 1024 * 1024
    return pl.pallas_call(
        functools.partial(_precompute_body, nhp, ntp, nrp),
        out_shape=(
            jax.ShapeDtypeStruct((nhp, nrp), jnp.float32),
            jax.ShapeDtypeStruct((1, ntp), jnp.float32),
            jax.ShapeDtypeStruct((1, nrp), jnp.float32),
        ),
    )(h_t, r_t, t_t)


def _combine_body(dim, acc_ref, v_ref):
    a = acc_ref[...]
    den = a[0:1, :] + a[2:3, :]
    num = a[1:2, :] + a[3:4, :]
    v = jnp.where(den > 0.0, num / den, 0.0)
    v_ref[...] = jnp.broadcast_to(v, (dim, v.shape[1]))


def _combine(acc, dim):
    b = acc.shape[1]
    return pl.pallas_call(
        functools.partial(_combine_body, dim),
        out_shape=jax.ShapeDtypeStruct((dim, b), jnp.float32),
    )(acc)


def _sc_kernel(n_tok, b, nrp,
               seg_hbm, r_hbm, t_hbm, h_hbm, m_hbm, tsum_hbm, rsum_hbm,
               acc_hbm,
               h_v, tsum_v, rsum_v, seg_l, r_l, t_l,
               seg_v, idx_v, s_v, ex_v, exd_v, zero_v,
               den_sp, num_sp,
               sem_in, sem_in2, sem_ga, sem_gb, sem_s):
    cid = lax.axis_index("c")
    sid = lax.axis_index("s")
    tid = cid * NS + sid
    tokens_per_w = n_tok // (NC * NS)
    rows_per_w = tokens_per_w // ROW
    bpw = b // NS
    tok0 = tid * tokens_per_w
    grp = 8
    ngrp = rows_per_w // grp

    copies_a = [
        (seg_hbm.at[pl.ds(tok0, tokens_per_w)], seg_l),
        (r_hbm.at[pl.ds(tok0, tokens_per_w)], r_l),
        (h_hbm, h_v),
    ]
    copies_b = [
        (t_hbm.at[pl.ds(tok0, tokens_per_w)], t_l),
        (tsum_hbm, tsum_v),
        (rsum_hbm, rsum_v),
    ]
    for src, dst in copies_a:
        pltpu.async_copy(src, dst, sem_in)
    for src, dst in copies_b:
        pltpu.async_copy(src, dst, sem_in2)

    @pl.loop(0, bpw // L)
    def _zero(k):
        zero_v[pl.ds(k * L, L)] = jnp.zeros((L,), jnp.float32)

    pltpu.sync_copy(zero_v, den_sp.at[pl.ds(sid * bpw, bpw)])
    pltpu.sync_copy(zero_v, num_sp.at[pl.ds(sid * bpw, bpw)])

    for src, dst in copies_a:
        pltpu.make_async_copy(src, dst, sem_in).wait()

    def _mk_idx(g):
        @pl.loop(g * grp, g * grp + grp)
        def _idx(j):
            for u in range(ROW // L):
                sl = pl.ds(u * L, L)
                fl = pl.ds(j * ROW + u * L, L)
                seg16 = seg_l[fl]
                r16 = r_l[fl]
                h16 = plsc.load_gather(h_v, [seg16])
                seg_v[j, sl] = seg16
                idx_v[j, sl] = h16 * nrp + r16

    gsems = (sem_ga, sem_gb)

    def _fire_gathers(g, sem):
        @pl.loop(g * grp, g * grp + grp)
        def _f(j):
            pltpu.async_copy(m_hbm.at[idx_v.at[j]], s_v.at[j], sem)

    def _drain_gathers(g, sem):
        @pl.loop(g * grp, g * grp + grp)
        def _d(j):
            pltpu.make_async_copy(m_hbm.at[idx_v.at[j]], s_v.at[j], sem).wait()

    _mk_idx(0)
    _fire_gathers(0, gsems[0])
    if ngrp > 1:
        _mk_idx(1)

    for src, dst in copies_b:
        pltpu.make_async_copy(src, dst, sem_in2).wait()

    plsc.subcore_barrier()

    for g in range(ngrp):
        if g + 1 < ngrp:
            _fire_gathers(g + 1, gsems[(g + 1) % 2])
        _drain_gathers(g, gsems[g % 2])

        @pl.loop(g * grp, g * grp + grp)
        def _compute(j):
            for u in range(ROW // L):
                sl = pl.ds(u * L, L)
                fl = pl.ds(j * ROW + u * L, L)
                s16 = s_v[j, sl]
                t16 = t_l[fl]
                r16 = r_l[fl]
                d16 = (plsc.load_gather(tsum_v, [t16])
                       - plsc.load_gather(rsum_v, [r16]))
                ex16 = jnp.exp(s16)
                ex_v[j, sl] = ex16
                exd_v[j, sl] = ex16 * d16

        @pl.loop(g * grp, g * grp + grp)
        def _scatter(j):
            pltpu.async_copy(ex_v.at[j], den_sp.at[seg_v.at[j]], sem_s,
                             add=True)
            pltpu.async_copy(exd_v.at[j], num_sp.at[seg_v.at[j]], sem_s,
                             add=True)

        if g + 2 < ngrp:
            _mk_idx(g + 2)

    @pl.loop(0, rows_per_w)
    def _drain_s(j):
        pltpu.make_async_copy(ex_v.at[j], den_sp.at[seg_v.at[j]],
                              sem_s).wait()
        pltpu.make_async_copy(exd_v.at[j], num_sp.at[seg_v.at[j]],
                              sem_s).wait()

    plsc.subcore_barrier()

    pltpu.sync_copy(den_sp.at[pl.ds(sid * bpw, bpw)],
                    acc_hbm.at[2 * cid, pl.ds(sid * bpw, bpw)])
    pltpu.sync_copy(num_sp.at[pl.ds(sid * bpw, bpw)],
                    acc_hbm.at[2 * cid + 1, pl.ds(sid * bpw, bpw)])


def _sc_run(seg, r_flat, t_flat, h, m_flat, tsum, rsum):
    n_tok = seg.shape[0]
    b = h.shape[0]
    ntp = tsum.shape[0]
    nrp = rsum.shape[0]
    bpw = b // NS
    tokens_per_w = n_tok // (NC * NS)
    rows_per_w = tokens_per_w // ROW
    mesh = plsc.VectorSubcoreMesh(
        core_axis_name="c", subcore_axis_name="s", num_cores=NC)
    grid_kernel = pl.kernel(
        functools.partial(_sc_kernel, n_tok, b, nrp),
        out_type=jax.ShapeDtypeStruct((4, b), jnp.float32),
        mesh=mesh,
        compiler_params=pltpu.CompilerParams(needs_layout_passes=False),
        scratch_types=[
            pltpu.VMEM((b,), jnp.int32),
            pltpu.VMEM((ntp,), jnp.float32),
            pltpu.VMEM((nrp,), jnp.float32),
            pltpu.VMEM((tokens_per_w,), jnp.int32),
            pltpu.VMEM((tokens_per_w,), jnp.int32),
            pltpu.VMEM((tokens_per_w,), jnp.int32),
            pltpu.VMEM((rows_per_w, ROW), jnp.int32),
            pltpu.VMEM((rows_per_w, ROW), jnp.int32),
            pltpu.VMEM((rows_per_w, ROW), jnp.float32),
            pltpu.VMEM((rows_per_w, ROW), jnp.float32),
            pltpu.VMEM((rows_per_w, ROW), jnp.float32),
            pltpu.VMEM((bpw,), jnp.float32),
            pltpu.VMEM_SHARED((b,), jnp.float32),
            pltpu.VMEM_SHARED((b,), jnp.float32),
            pltpu.SemaphoreType.DMA,
            pltpu.SemaphoreType.DMA,
            pltpu.SemaphoreType.DMA,
            pltpu.SemaphoreType.DMA,
            pltpu.SemaphoreType.DMA,
        ],
    )
    return grid_kernel(seg, r_flat, t_flat, h, m_flat, tsum, rsum)


def kernel(h, r_flat, t_flat, segment_ids, H_table, R_table, T_table):
    b = h.shape[0]
    dim = H_table.shape[1]
    m, ts, rs = _precompute(H_table.T, R_table.T, T_table.T)

    acc = _sc_run(segment_ids, r_flat, t_flat, h, m.reshape(-1),
                  ts.reshape(-1), rs.reshape(-1))
    return _combine(acc, dim).T
